# baseline (speedup 1.0000x reference)
import jax
import jax.numpy as jnp
from jax import lax
from jax.experimental import pallas as pl
from jax.experimental.pallas import tpu as pltpu
from jax.experimental.pallas import tpu_sc as plsc

_B, _N, _M = 8, 4096, 256
_KEYMASK = ~0xFFF
_IDXMASK = 0xFFF
_KEYMAX = 0x7FFFFFFF

_NW = 32
_SLOTS = _B * _M * 2
_SPW = _SLOTS // _NW
_LANES = 16


def _tc_body(shapeT_ref, skel_ref, iota_ref, out_cd, out_idx, cda):
    b = pl.program_id(0)

    p3 = shapeT_ref[0, 0:3, :]
    sk = skel_ref[0]

    cross = jnp.dot(sk, p3, precision=lax.Precision.HIGHEST,
                    preferred_element_type=jnp.float32)
    p2 = jnp.sum(p3 * p3, axis=0, keepdims=True)
    s2 = jnp.sum(sk * sk, axis=1, keepdims=True)
    d2m = jnp.maximum((s2 - 2.0 * cross) + p2, 0.0)

    c1 = jnp.min(d2m, axis=0, keepdims=True)
    cd_part = jnp.sum(jnp.sqrt(c1 + 1e-12), keepdims=True).reshape(1, 1)

    key = (lax.bitcast_convert_type(d2m, jnp.int32) & _KEYMASK) | iota_ref[...]
    bk1 = jnp.min(key, axis=1, keepdims=True)
    mk = jnp.where(key == bk1, _KEYMAX, key)
    bk2 = jnp.min(mk, axis=1, keepdims=True)

    d2best = lax.bitcast_convert_type(bk1 & _KEYMASK, jnp.float32)
    cd2v = jnp.sum(jnp.sqrt(d2best + 1e-12), keepdims=True).reshape(1, 1)
    total = cd_part + cd2v

    out_idx[0, :, 0:1] = bk1 & _IDXMASK
    out_idx[0, :, 1:2] = bk2 & _IDXMASK

    @pl.when(b == 0)
    def _first():
        cda[...] = total

    @pl.when(b != 0)
    def _rest():
        cda[...] = cda[...] + total

    @pl.when(b == _B - 1)
    def _emit():
        out_cd[...] = cda[...]


def _tc_call(shapeT, skel_xyz):
    iota_row = lax.broadcasted_iota(jnp.int32, (1, _N), 1)
    return pl.pallas_call(
        _tc_body,
        grid=(_B,),
        in_specs=[
            pl.BlockSpec((1, 6, _N), lambda b: (b, 0, 0)),
            pl.BlockSpec((1, _M, 3), lambda b: (b, 0, 0)),
            pl.BlockSpec((1, _N), lambda b: (0, 0)),
        ],
        out_specs=[
            pl.BlockSpec((1, 1), lambda b: (0, 0)),
            pl.BlockSpec((1, _M, 2), lambda b: (b, 0, 0)),
        ],
        out_shape=[
            jax.ShapeDtypeStruct((1, 1), jnp.float32),
            jax.ShapeDtypeStruct((_B, _M, 2), jnp.int32),
        ],
        scratch_shapes=[
            pltpu.VMEM((1, 1), jnp.float32),
        ],
    )(shapeT, skel_xyz, iota_row)


def _sc_body(shape_hbm, idx_hbm, nori_hbm, out_hbm,
             pts_v, idx_v, nori_v, acc_v, sem):
    cid = lax.axis_index("c")
    sid = lax.axis_index("s")
    wid = cid * 16 + sid
    b = wid >> 2
    m0 = (wid & 3) * (_SPW // 2)
    stage = pltpu.async_copy(
        shape_hbm.at[pl.ds(b * (_N * 6), _N * 6)], pts_v, sem)
    pltpu.sync_copy(idx_hbm.at[wid], idx_v)
    pltpu.sync_copy(nori_hbm.at[pl.ds(b * (_M * 3), _M * 3)], nori_v)
    stage.wait()
    acc = jnp.zeros((_LANES,), jnp.float32)
    ids0 = lax.broadcasted_iota(jnp.int32, (_LANES,), 0)
    half = lax.shift_right_logical(ids0, 1)
    for j in range(_SPW // _LANES):
        sl = pl.ds(j * _LANES, _LANES)
        r = idx_v[sl] * 6 + 3
        nx = plsc.load_gather(pts_v, [r])
        ny = plsc.load_gather(pts_v, [r + 1])
        nz = plsc.load_gather(pts_v, [r + 2])
        mi = (m0 + 8 * j + half) * 3
        ox = plsc.load_gather(nori_v, [mi])
        oy = plsc.load_gather(nori_v, [mi + 1])
        oz = plsc.load_gather(nori_v, [mi + 2])
        acc = acc + jnp.abs(nx * ox + ny * oy + nz * oz)
    acc_v[...] = acc
    pltpu.sync_copy(acc_v, out_hbm.at[wid])


def _sc_call(shape_flat, idx_w, nori_flat):
    return pl.kernel(
        _sc_body,
        out_type=jax.ShapeDtypeStruct((_NW, _LANES), jnp.float32),
        mesh=plsc.VectorSubcoreMesh(core_axis_name="c", subcore_axis_name="s"),
        compiler_params=pltpu.CompilerParams(needs_layout_passes=False),
        scratch_types=[
            pltpu.VMEM((_N * 6,), jnp.float32),
            pltpu.VMEM((_SPW,), jnp.int32),
            pltpu.VMEM((_M * 3,), jnp.float32),
            pltpu.VMEM((_LANES,), jnp.float32),
            pltpu.SemaphoreType.DMA,
        ],
    )(shape_flat, idx_w, nori_flat)


def kernel(shape_xyz, skel_xyz, skel_nori):
    shapeT = jnp.transpose(shape_xyz, (0, 2, 1))
    cd_raw, idx = _tc_call(shapeT, skel_xyz)

    idx_w = idx.reshape(_NW, _SPW)
    shape_flat = shape_xyz.reshape(_B * _N * 6)
    nori_flat = skel_nori.reshape(_B * _M * 3)

    parts = _sc_call(shape_flat, idx_w, nori_flat)
    return cd_raw[0, 0] * 1e-4 + 0.001 * (jnp.sum(parts) / (2.0 * _B))


# speedup vs baseline: 1.0942x; 1.0942x over previous
import jax
import jax.numpy as jnp
from jax import lax
from jax.experimental import pallas as pl
from jax.experimental.pallas import tpu as pltpu
from jax.experimental.pallas import tpu_sc as plsc

_B, _N, _M = 8, 4096, 256
_KEYMASK = ~0xFFF
_IDXMASK = 0xFFF
_KEYMAX = 0x7FFFFFFF

_NW = 32
_SLOTS = _B * _M * 2
_SPW = _SLOTS // _NW
_LANES = 16


def _tc_body(shapeT_ref, skel_ref, iota_ref, out_cd, out_idx, cda):
    b = pl.program_id(0)

    pt = shapeT_ref[0]
    px, py, pz = pt[0:1, :], pt[1:2, :], pt[2:3, :]
    sk = skel_ref[0]
    sx, sy, sz = sk[:, 0:1], sk[:, 1:2], sk[:, 2:3]

    dxx = sx - px
    dyy = sy - py
    dzz = sz - pz
    d2m = dxx * dxx + dyy * dyy + dzz * dzz

    c1 = jnp.min(d2m, axis=0, keepdims=True)
    cd_part = jnp.sum(jnp.sqrt(c1 + 1e-12), keepdims=True).reshape(1, 1)

    key = (lax.bitcast_convert_type(d2m, jnp.int32) & _KEYMASK) | iota_ref[...]
    bk1 = jnp.min(key, axis=1, keepdims=True)
    mk = jnp.where(key == bk1, _KEYMAX, key)
    bk2 = jnp.min(mk, axis=1, keepdims=True)

    d2best = lax.bitcast_convert_type(bk1 & _KEYMASK, jnp.float32)
    cd2v = jnp.sum(jnp.sqrt(d2best + 1e-12), keepdims=True).reshape(1, 1)
    total = cd_part + cd2v

    out_idx[0, :, 0:1] = bk1 & _IDXMASK
    out_idx[0, :, 1:2] = bk2 & _IDXMASK

    @pl.when(b == 0)
    def _first():
        cda[...] = total

    @pl.when(b != 0)
    def _rest():
        cda[...] = cda[...] + total

    @pl.when(b == _B - 1)
    def _emit():
        out_cd[...] = cda[...]


def _tc_call(shapeT, skel_xyz):
    iota_row = lax.broadcasted_iota(jnp.int32, (1, _N), 1)
    return pl.pallas_call(
        _tc_body,
        grid=(_B,),
        in_specs=[
            pl.BlockSpec((1, 6, _N), lambda b: (b, 0, 0)),
            pl.BlockSpec((1, _M, 3), lambda b: (b, 0, 0)),
            pl.BlockSpec((1, _N), lambda b: (0, 0)),
        ],
        out_specs=[
            pl.BlockSpec((1, 1), lambda b: (0, 0)),
            pl.BlockSpec((1, _M, 2), lambda b: (b, 0, 0)),
        ],
        out_shape=[
            jax.ShapeDtypeStruct((1, 1), jnp.float32),
            jax.ShapeDtypeStruct((_B, _M, 2), jnp.int32),
        ],
        scratch_shapes=[
            pltpu.VMEM((1, 1), jnp.float32),
        ],
    )(shapeT, skel_xyz, iota_row)


def _sc_body(shape_hbm, idx_hbm, nori_hbm, out_hbm,
             pts_v, idx_v, nori_v, acc_v, sem):
    cid = lax.axis_index("c")
    sid = lax.axis_index("s")
    wid = cid * 16 + sid
    b = wid >> 2
    m0 = (wid & 3) * (_SPW // 2)
    stage = pltpu.async_copy(
        shape_hbm.at[pl.ds(b * (_N * 6), _N * 6)], pts_v, sem)
    pltpu.sync_copy(idx_hbm.at[wid], idx_v)
    pltpu.sync_copy(nori_hbm.at[pl.ds(b * (_M * 3), _M * 3)], nori_v)
    stage.wait()
    acc = jnp.zeros((_LANES,), jnp.float32)
    ids0 = lax.broadcasted_iota(jnp.int32, (_LANES,), 0)
    half = lax.shift_right_logical(ids0, 1)
    for j in range(_SPW // _LANES):
        sl = pl.ds(j * _LANES, _LANES)
        r = idx_v[sl] * 6 + 3
        nx = plsc.load_gather(pts_v, [r])
        ny = plsc.load_gather(pts_v, [r + 1])
        nz = plsc.load_gather(pts_v, [r + 2])
        mi = (m0 + 8 * j + half) * 3
        ox = plsc.load_gather(nori_v, [mi])
        oy = plsc.load_gather(nori_v, [mi + 1])
        oz = plsc.load_gather(nori_v, [mi + 2])
        acc = acc + jnp.abs(nx * ox + ny * oy + nz * oz)
    acc_v[...] = acc
    pltpu.sync_copy(acc_v, out_hbm.at[wid])


def _sc_call(shape_flat, idx_w, nori_flat):
    return pl.kernel(
        _sc_body,
        out_type=jax.ShapeDtypeStruct((_NW, _LANES), jnp.float32),
        mesh=plsc.VectorSubcoreMesh(core_axis_name="c", subcore_axis_name="s"),
        compiler_params=pltpu.CompilerParams(needs_layout_passes=False),
        scratch_types=[
            pltpu.VMEM((_N * 6,), jnp.float32),
            pltpu.VMEM((_SPW,), jnp.int32),
            pltpu.VMEM((_M * 3,), jnp.float32),
            pltpu.VMEM((_LANES,), jnp.float32),
            pltpu.SemaphoreType.DMA,
        ],
    )(shape_flat, idx_w, nori_flat)


def kernel(shape_xyz, skel_xyz, skel_nori):
    shapeT = jnp.transpose(shape_xyz, (0, 2, 1))
    cd_raw, idx = _tc_call(shapeT, skel_xyz)

    idx_w = idx.reshape(_NW, _SPW)
    shape_flat = shape_xyz.reshape(_B * _N * 6)
    nori_flat = skel_nori.reshape(_B * _M * 3)

    parts = _sc_call(shape_flat, idx_w, nori_flat)
    return cd_raw[0, 0] * 1e-4 + 0.001 * (jnp.sum(parts) / (2.0 * _B))


# speedup vs baseline: 1.1182x; 1.0220x over previous
import jax
import jax.numpy as jnp
from jax import lax
from jax.experimental import pallas as pl
from jax.experimental.pallas import tpu as pltpu
from jax.experimental.pallas import tpu_sc as plsc

_B, _N, _M = 8, 4096, 256
_KEYMASK = ~0xFFF
_IDXMASK = 0xFFF
_KEYMAX = 0x7FFFFFFF

_NW = 32
_SLOTS = _B * _M * 2
_SPW = _SLOTS // _NW
_LANES = 16


def _tc_body(shapeT_ref, skel_ref, out_cd, out_idx, cda):
    b = pl.program_id(0)

    pt = shapeT_ref[0]
    px, py, pz = pt[0:1, :], pt[1:2, :], pt[2:3, :]
    sk = skel_ref[0]
    sx, sy, sz = sk[:, 0:1], sk[:, 1:2], sk[:, 2:3]

    dxx = sx - px
    dyy = sy - py
    dzz = sz - pz
    d2m = dxx * dxx + dyy * dyy + dzz * dzz

    c1 = jnp.min(d2m, axis=0, keepdims=True)
    cd_part = jnp.sum(jnp.sqrt(c1 + 1e-12), keepdims=True).reshape(1, 1)

    ri = lax.broadcasted_iota(jnp.int32, (_M, _N), 1)
    key = (lax.bitcast_convert_type(d2m, jnp.int32) & _KEYMASK) | ri
    bk1 = jnp.min(key, axis=1, keepdims=True)
    mk = jnp.where(key == bk1, _KEYMAX, key)
    bk2 = jnp.min(mk, axis=1, keepdims=True)

    d2best = lax.bitcast_convert_type(bk1 & _KEYMASK, jnp.float32)
    cd2v = jnp.sum(jnp.sqrt(d2best + 1e-12), keepdims=True).reshape(1, 1)
    total = cd_part + cd2v

    out_idx[0, :, 0:1] = bk1 & _IDXMASK
    out_idx[0, :, 1:2] = bk2 & _IDXMASK

    @pl.when(b == 0)
    def _first():
        cda[...] = total

    @pl.when(b != 0)
    def _rest():
        cda[...] = cda[...] + total

    @pl.when(b == _B - 1)
    def _emit():
        out_cd[...] = cda[...]


def _tc_call(shapeT, skel_xyz):
    return pl.pallas_call(
        _tc_body,
        grid=(_B,),
        in_specs=[
            pl.BlockSpec((1, 6, _N), lambda b: (b, 0, 0)),
            pl.BlockSpec((1, _M, 3), lambda b: (b, 0, 0)),
        ],
        out_specs=[
            pl.BlockSpec((1, 1), lambda b: (0, 0)),
            pl.BlockSpec((1, _M, 2), lambda b: (b, 0, 0)),
        ],
        out_shape=[
            jax.ShapeDtypeStruct((1, 1), jnp.float32),
            jax.ShapeDtypeStruct((_B, _M, 2), jnp.int32),
        ],
        scratch_shapes=[
            pltpu.VMEM((1, 1), jnp.float32),
        ],
    )(shapeT, skel_xyz)


def _sc_body(shape_hbm, idx_hbm, nori_hbm, out_hbm,
             pts_v, idx_v, nori_v, acc_v, sem):
    cid = lax.axis_index("c")
    sid = lax.axis_index("s")
    wid = cid * 16 + sid
    b = wid >> 2
    m0 = (wid & 3) * (_SPW // 2)
    stage = pltpu.async_copy(
        shape_hbm.at[pl.ds(b * (_N * 6), _N * 6)], pts_v, sem)
    pltpu.sync_copy(idx_hbm.at[wid], idx_v)
    pltpu.sync_copy(nori_hbm.at[pl.ds(b * (_M * 3), _M * 3)], nori_v)
    stage.wait()
    acc = jnp.zeros((_LANES,), jnp.float32)
    ids0 = lax.broadcasted_iota(jnp.int32, (_LANES,), 0)
    half = lax.shift_right_logical(ids0, 1)
    for j in range(_SPW // _LANES):
        sl = pl.ds(j * _LANES, _LANES)
        r = idx_v[sl] * 6 + 3
        nx = plsc.load_gather(pts_v, [r])
        ny = plsc.load_gather(pts_v, [r + 1])
        nz = plsc.load_gather(pts_v, [r + 2])
        mi = (m0 + 8 * j + half) * 3
        ox = plsc.load_gather(nori_v, [mi])
        oy = plsc.load_gather(nori_v, [mi + 1])
        oz = plsc.load_gather(nori_v, [mi + 2])
        acc = acc + jnp.abs(nx * ox + ny * oy + nz * oz)
    acc_v[...] = acc
    pltpu.sync_copy(acc_v, out_hbm.at[wid])


def _sc_call(shape_flat, idx_w, nori_flat):
    return pl.kernel(
        _sc_body,
        out_type=jax.ShapeDtypeStruct((_NW, _LANES), jnp.float32),
        mesh=plsc.VectorSubcoreMesh(core_axis_name="c", subcore_axis_name="s"),
        compiler_params=pltpu.CompilerParams(needs_layout_passes=False),
        scratch_types=[
            pltpu.VMEM((_N * 6,), jnp.float32),
            pltpu.VMEM((_SPW,), jnp.int32),
            pltpu.VMEM((_M * 3,), jnp.float32),
            pltpu.VMEM((_LANES,), jnp.float32),
            pltpu.SemaphoreType.DMA,
        ],
    )(shape_flat, idx_w, nori_flat)


def kernel(shape_xyz, skel_xyz, skel_nori):
    shapeT = jnp.transpose(shape_xyz, (0, 2, 1))
    cd_raw, idx = _tc_call(shapeT, skel_xyz)

    idx_w = idx.reshape(_NW, _SPW)
    shape_flat = shape_xyz.reshape(_B * _N * 6)
    nori_flat = skel_nori.reshape(_B * _M * 3)

    parts = _sc_call(shape_flat, idx_w, nori_flat)
    return cd_raw[0, 0] * 1e-4 + 0.001 * (jnp.sum(parts) / (2.0 * _B))
